# baseline (speedup 1.0000x reference)
import functools

import numpy as np

import jax
import jax.numpy as jnp
from jax import lax
from jax.experimental import pallas as pl
from jax.experimental.pallas import tpu as pltpu
from jax.experimental.pallas import tpu_sc as plsc

B = 128
V = 100000
L = 16
NC, NS = 2, 16
NG = B // 8
TILE = 128
HTILES = 390
HCOLS = HTILES * TILE
TAIL0 = 2 * HCOLS
TAILC = V - TAIL0
CT = 13
CW = CT * TILE
NCHUNK = HTILES // CT
CIT = CW // L
TIT = TAILC // L


def _threefry2x32(k1, k2, x0, x1):
    u32 = np.uint32
    R0 = (13, 15, 26, 6)
    R1 = (17, 29, 16, 24)
    ks = (u32(k1), u32(k2), u32(k1) ^ u32(k2) ^ u32(0x1BD11BDA))
    x0 = (x0 + ks[0]).astype(u32)
    x1 = (x1 + ks[1]).astype(u32)

    def rounds(x0, x1, rs):
        for r in rs:
            x0 = (x0 + x1).astype(u32)
            x1 = ((x1 << u32(r)) | (x1 >> u32(32 - r))).astype(u32) ^ x0
        return x0, x1

    for i, (rs, a, b) in enumerate(
            [(R0, 1, 2), (R1, 2, 0), (R0, 0, 1), (R1, 1, 2), (R0, 2, 0)]):
        x0, x1 = rounds(x0, x1, rs)
        x0 = (x0 + ks[a]).astype(u32)
        x1 = (x1 + ks[b] + u32(i + 1)).astype(u32)
    return x0, x1


def _gumbel_const():
    n = B * V
    idx = np.arange(n, dtype=np.uint64)
    hi = (idx >> np.uint64(32)).astype(np.uint32)
    lo = (idx & np.uint64(0xFFFFFFFF)).astype(np.uint32)
    b1, b2 = _threefry2x32(np.uint32(0), np.uint32(42), hi, lo)
    bits = b1 ^ b2
    fb = (bits >> np.uint32(9)) | np.float32(1.0).view(np.uint32)
    u = fb.view(np.float32) - np.float32(1.0)
    noise = (-np.log1p(-u.astype(np.float64))).astype(np.float32)
    noise = np.maximum(noise, np.float32(1e-10))
    g = np.log(noise.astype(np.float64)).astype(np.float32)
    g = g.reshape(B, V)
    ntiles = (V + TILE - 1) // TILE
    gp = np.zeros((B, ntiles * TILE), np.float32)
    gp[:, :V] = g
    return np.ascontiguousarray(
        gp.reshape(NG, 8, ntiles, TILE).transpose(0, 2, 1, 3))


_G = _gumbel_const()

try:
    _G_OP = jax.new_ref(jnp.asarray(_G))
except Exception:
    _G_OP = _G

_mesh = plsc.VectorSubcoreMesh(core_axis_name="c", subcore_axis_name="s")

VPAD = (V + TILE - 1) // TILE * TILE

def _pad_logits(logits):
    return jnp.pad(logits, ((0, 0), (0, VPAD - V)))


@functools.partial(
    pl.kernel,
    out_type=jax.ShapeDtypeStruct((NC * NS * L,), jnp.int32),
    mesh=_mesh,
    compiler_params=pltpu.CompilerParams(needs_layout_passes=False),
    scratch_types=[
        pltpu.VMEM((4, 8, CW), jnp.float32),
        pltpu.VMEM((4, CT, 8, TILE), jnp.float32),
        pltpu.VMEM((8, 2 * TILE), jnp.float32),
        pltpu.VMEM((2, 8, TILE), jnp.float32),
        pltpu.VMEM((L,), jnp.float32),
        pltpu.VMEM((L,), jnp.float32),
        pltpu.VMEM((L,), jnp.int32),
        pltpu.VMEM((L,), jnp.float32),
        pltpu.VMEM((L,), jnp.int32),
        pltpu.VMEM((L,), jnp.int32),
        pltpu.VMEM_SHARED((NS * L,), jnp.float32),
        pltpu.VMEM_SHARED((NS * L,), jnp.int32),
        pltpu.SemaphoreType.DMA,
        pltpu.SemaphoreType.DMA,
        pltpu.SemaphoreType.DMA,
        pltpu.SemaphoreType.DMA,
        pltpu.SemaphoreType.DMA,
    ],
)
def _sampler(logits_hbm, params_hbm, g_hbm, out_hbm,
             lbuf, gbuf, ltail, gtail, pbuf, mvbuf, mibuf, pvbuf, pibuf,
             tokbuf, shv, shi, sem0, sem1, sem2, sem3, sems):
    c = lax.axis_index("c")
    s = lax.axis_index("s")
    w = c * NS + s
    g = c * 8 + s // 2
    h = s % 2
    row0 = pl.multiple_of(g * 8, 8)
    col_h = pl.multiple_of(h * HCOLS, TILE)
    semslot = (sem0, sem1, sem2, sem3)

    woff = pl.multiple_of(w * L, 8)
    pltpu.sync_copy(params_hbm.at[pl.ds(woff, L)], pbuf)
    pvec = pbuf[...]
    tpv = [jnp.full((L,), pvec[r], jnp.float32) for r in range(8)]
    bsv = [jnp.full((L,), pvec[8 + r], jnp.float32) for r in range(8)]

    lanes = lax.iota(jnp.int32, L)

    tile_h = h * HTILES

    def start(chunk, slot):
        sem = semslot[slot]
        cl = pltpu.async_copy(
            logits_hbm.at[pl.ds(row0, 8), pl.ds(col_h + chunk * CW, CW)],
            lbuf.at[slot], sem)
        cg = pltpu.async_copy(
            g_hbm.at[g, pl.ds(tile_h + chunk * CT, CT)], gbuf.at[slot], sem)
        return cl, cg

    tl = pltpu.async_copy(
        logits_hbm.at[pl.ds(row0, 8), pl.ds(TAIL0, 2 * TILE)], ltail, sems)
    tg = pltpu.async_copy(
        g_hbm.at[g, pl.ds(2 * HTILES, 2)], gtail, sems)

    best = [jnp.full((L,), -jnp.inf, jnp.float32) for _ in range(8)]
    bidx = [jnp.zeros((L,), jnp.int32) for _ in range(8)]

    def make_body(lref, gref, colbase):
        def body(i, carry):
            bs_ = list(carry[:8])
            bi_ = list(carry[8:])
            t = i >> 3
            joff = (i & 7) * L
            off = i * L
            idx = lanes + (colbase + off)
            for r in range(8):
                v = lref[r, pl.ds(off, L)]
                gg = gref[t, r, pl.ds(joff, L)]
                u = v / tpv[r] - gg * bsv[r]
                m = u > bs_[r]
                bs_[r] = jnp.where(m, u, bs_[r])
                bi_[r] = jnp.where(m, idx, bi_[r])
            return tuple(bs_) + tuple(bi_)
        return body

    NBUF = 4
    pend = [start(k, k) for k in range(NBUF - 1)]
    for chunk in range(NCHUNK):
        slot = chunk % NBUF
        cl, cg = pend.pop(0)
        nxt = chunk + NBUF - 1
        if nxt < NCHUNK:
            pend.append(start(nxt, nxt % NBUF))
        cl.wait()
        cg.wait()
        carry = lax.fori_loop(
            0, CIT, make_body(lbuf.at[slot], gbuf.at[slot], col_h + chunk * CW),
            tuple(best) + tuple(bidx))
        best, bidx = list(carry[:8]), list(carry[8:])

    tl.wait()
    tg.wait()
    carry = tuple(best) + tuple(bidx)
    bs_ = list(carry[:8])
    bi_ = list(carry[8:])
    for i in range(TIT):
        t, j = divmod(i, 8)
        idx = lanes + (TAIL0 + i * L)
        for r in range(8):
            v = ltail[r, pl.ds(i * L, L)]
            gg = gtail[t, r, pl.ds(j * L, L)]
            u = v / tpv[r] - gg * bsv[r]
            m = u > bs_[r]
            bs_[r] = jnp.where(m, u, bs_[r])
            bi_[r] = jnp.where(m, idx, bi_[r])
    best, bidx = bs_, bi_

    mv = jnp.zeros((L,), jnp.float32)
    mi = jnp.zeros((L,), jnp.int32)
    for r in range(8):
        m = jnp.max(best[r])
        tok = jnp.min(jnp.where(best[r] == m, bidx[r], jnp.int32(2**31 - 1)))
        mv = jnp.where(lanes == r, m, mv)
        mi = jnp.where(lanes == r, tok, mi)
    mvbuf[...] = mv
    mibuf[...] = mi

    soff = pl.multiple_of(s * L, 8)
    pltpu.sync_copy(mvbuf, shv.at[pl.ds(soff, L)])
    pltpu.sync_copy(mibuf, shi.at[pl.ds(soff, L)])
    plsc.subcore_barrier()
    poff = pl.multiple_of((s + 1 - 2 * h) * L, 8)
    pltpu.sync_copy(shv.at[pl.ds(poff, L)], pvbuf)
    pltpu.sync_copy(shi.at[pl.ds(poff, L)], pibuf)
    pv = pvbuf[...]
    pi = pibuf[...]

    better = pv > mv
    tie = pv == mv
    toki = jnp.where(better, pi, jnp.where(tie, jnp.minimum(pi, mi), mi))
    tokbuf[...] = toki
    pltpu.sync_copy(tokbuf, out_hbm.at[pl.ds(woff, L)])


def kernel(logits, temperatures):
    greedy = temperatures == 0
    tp = jnp.where(greedy, jnp.ones_like(temperatures), temperatures)
    bs = jnp.where(greedy, 0.0, 1.0).astype(jnp.float32)
    gidx = (jnp.arange(NC * NS) // NS) * 8 + (jnp.arange(NC * NS) % NS) // 2
    params = jnp.concatenate(
        [tp.reshape(NG, 8)[gidx], bs.reshape(NG, 8)[gidx]], axis=1).reshape(-1)
    out = _sampler(_pad_logits(logits), params, _G_OP)
    return out.reshape(NC, 8, 2, L)[:, :, 0, :8].reshape(B)


# speedup vs baseline: 1.2315x; 1.2315x over previous
import functools

import numpy as np

import jax
import jax.numpy as jnp
from jax import lax
from jax.experimental import pallas as pl
from jax.experimental.pallas import tpu as pltpu
from jax.experimental.pallas import tpu_sc as plsc

B = 128
V = 100000
L = 16
NC, NS = 2, 16
NG = B // 8
TILE = 128
HTILES = 390
HCOLS = HTILES * TILE
TAIL0 = 2 * HCOLS
TAILC = V - TAIL0
CT = 13
CW = CT * TILE
NCHUNK = HTILES // CT
CIT = CW // L
TIT = TAILC // L


def _threefry2x32(k1, k2, x0, x1):
    u32 = np.uint32
    R0 = (13, 15, 26, 6)
    R1 = (17, 29, 16, 24)
    ks = (u32(k1), u32(k2), u32(k1) ^ u32(k2) ^ u32(0x1BD11BDA))
    x0 = (x0 + ks[0]).astype(u32)
    x1 = (x1 + ks[1]).astype(u32)

    def rounds(x0, x1, rs):
        for r in rs:
            x0 = (x0 + x1).astype(u32)
            x1 = ((x1 << u32(r)) | (x1 >> u32(32 - r))).astype(u32) ^ x0
        return x0, x1

    for i, (rs, a, b) in enumerate(
            [(R0, 1, 2), (R1, 2, 0), (R0, 0, 1), (R1, 1, 2), (R0, 2, 0)]):
        x0, x1 = rounds(x0, x1, rs)
        x0 = (x0 + ks[a]).astype(u32)
        x1 = (x1 + ks[b] + u32(i + 1)).astype(u32)
    return x0, x1


def _gumbel_const():
    n = B * V
    idx = np.arange(n, dtype=np.uint64)
    hi = (idx >> np.uint64(32)).astype(np.uint32)
    lo = (idx & np.uint64(0xFFFFFFFF)).astype(np.uint32)
    b1, b2 = _threefry2x32(np.uint32(0), np.uint32(42), hi, lo)
    bits = b1 ^ b2
    fb = (bits >> np.uint32(9)) | np.float32(1.0).view(np.uint32)
    u = fb.view(np.float32) - np.float32(1.0)
    noise = (-np.log1p(-u.astype(np.float64))).astype(np.float32)
    noise = np.maximum(noise, np.float32(1e-10))
    g = np.log(noise.astype(np.float64)).astype(np.float32)
    g = g.reshape(B, V)
    ntiles = (V + TILE - 1) // TILE
    gp = np.zeros((B, ntiles * TILE), np.float32)
    gp[:, :V] = g
    return np.ascontiguousarray(
        gp.reshape(NG, 8, ntiles, TILE).transpose(0, 2, 1, 3))


_G = _gumbel_const()

try:
    _G_OP = jax.new_ref(jnp.asarray(_G))
except Exception:
    _G_OP = _G

_mesh = plsc.VectorSubcoreMesh(core_axis_name="c", subcore_axis_name="s")

VPAD = (V + TILE - 1) // TILE * TILE

def _pad_logits(logits):
    return jnp.pad(logits, ((0, 0), (0, VPAD - V)))


@functools.partial(
    pl.kernel,
    out_type=jax.ShapeDtypeStruct((NC * NS * L,), jnp.int32),
    mesh=_mesh,
    compiler_params=pltpu.CompilerParams(needs_layout_passes=False),
    scratch_types=[
        pltpu.VMEM((4, 8, CW), jnp.float32),
        pltpu.VMEM((4, CT, 8, TILE), jnp.float32),
        pltpu.VMEM((8, TAILC), jnp.float32),
        pltpu.VMEM((2, 8, TILE), jnp.float32),
        pltpu.VMEM((L,), jnp.float32),
        pltpu.VMEM((L,), jnp.float32),
        pltpu.VMEM((L,), jnp.int32),
        pltpu.VMEM((L,), jnp.float32),
        pltpu.VMEM((L,), jnp.int32),
        pltpu.VMEM((L,), jnp.int32),
        pltpu.VMEM_SHARED((NS * L,), jnp.float32),
        pltpu.VMEM_SHARED((NS * L,), jnp.int32),
        pltpu.SemaphoreType.DMA,
        pltpu.SemaphoreType.DMA,
        pltpu.SemaphoreType.DMA,
        pltpu.SemaphoreType.DMA,
        pltpu.SemaphoreType.DMA,
    ],
)
def _sampler(logits_hbm, params_hbm, g_hbm, out_hbm,
             lbuf, gbuf, ltail, gtail, pbuf, mvbuf, mibuf, pvbuf, pibuf,
             tokbuf, shv, shi, sem0, sem1, sem2, sem3, sems):
    c = lax.axis_index("c")
    s = lax.axis_index("s")
    w = c * NS + s
    g = c * 8 + s // 2
    h = s % 2
    row0 = pl.multiple_of(g * 8, 8)
    col_h = pl.multiple_of(h * HCOLS, TILE)
    semslot = (sem0, sem1, sem2, sem3)

    woff = pl.multiple_of(w * L, 8)
    pltpu.sync_copy(params_hbm.at[pl.ds(woff, L)], pbuf)
    pvec = pbuf[...]
    wv = [jnp.full((L,), pvec[r], jnp.float32) for r in range(8)]

    lanes = lax.iota(jnp.int32, L)

    tile_h = h * HTILES

    def start(chunk, slot):
        sem = semslot[slot]
        cl = pltpu.async_copy(
            logits_hbm.at[pl.ds(row0, 8), pl.ds(col_h + chunk * CW, CW)],
            lbuf.at[slot], sem)
        cg = pltpu.async_copy(
            g_hbm.at[g, pl.ds(tile_h + chunk * CT, CT)], gbuf.at[slot], sem)
        return cl, cg

    tl = pltpu.async_copy(
        logits_hbm.at[pl.ds(row0, 8), pl.ds(TAIL0, TAILC)], ltail, sems)
    tg = pltpu.async_copy(
        g_hbm.at[g, pl.ds(2 * HTILES, 2)], gtail, sems)

    best = [jnp.full((L,), -jnp.inf, jnp.float32) for _ in range(8)]
    bidx = [jnp.zeros((L,), jnp.int32) for _ in range(8)]

    def make_body(lref, gref, colbase):
        def body(i, carry):
            bs_ = list(carry[:8])
            bi_ = list(carry[8:])
            t = i >> 3
            joff = (i & 7) * L
            off = i * L
            idx = lanes + (colbase + off)
            for r in range(8):
                v = lref[r, pl.ds(off, L)]
                gg = gref[t, r, pl.ds(joff, L)]
                u = v - gg * wv[r]
                m = u > bs_[r]
                bs_[r] = jnp.where(m, u, bs_[r])
                bi_[r] = jnp.where(m, idx, bi_[r])
            return tuple(bs_) + tuple(bi_)
        return body

    NBUF = 4
    pend = [start(k, k) for k in range(NBUF - 1)]
    for chunk in range(NCHUNK):
        slot = chunk % NBUF
        cl, cg = pend.pop(0)
        nxt = chunk + NBUF - 1
        if nxt < NCHUNK:
            pend.append(start(nxt, nxt % NBUF))
        cl.wait()
        cg.wait()
        carry = lax.fori_loop(
            0, CIT, make_body(lbuf.at[slot], gbuf.at[slot], col_h + chunk * CW),
            tuple(best) + tuple(bidx))
        best, bidx = list(carry[:8]), list(carry[8:])

    tl.wait()
    tg.wait()
    carry = tuple(best) + tuple(bidx)
    bs_ = list(carry[:8])
    bi_ = list(carry[8:])
    for i in range(TIT):
        t, j = divmod(i, 8)
        idx = lanes + (TAIL0 + i * L)
        for r in range(8):
            v = ltail[r, pl.ds(i * L, L)]
            gg = gtail[t, r, pl.ds(j * L, L)]
            u = v - gg * wv[r]
            m = u > bs_[r]
            bs_[r] = jnp.where(m, u, bs_[r])
            bi_[r] = jnp.where(m, idx, bi_[r])
    best, bidx = bs_, bi_

    mv = jnp.zeros((L,), jnp.float32)
    mi = jnp.zeros((L,), jnp.int32)
    for r in range(8):
        m = jnp.max(best[r])
        tok = jnp.min(jnp.where(best[r] == m, bidx[r], jnp.int32(2**31 - 1)))
        mv = jnp.where(lanes == r, m, mv)
        mi = jnp.where(lanes == r, tok, mi)
    mvbuf[...] = mv
    mibuf[...] = mi

    soff = pl.multiple_of(s * L, 8)
    pltpu.sync_copy(mvbuf, shv.at[pl.ds(soff, L)])
    pltpu.sync_copy(mibuf, shi.at[pl.ds(soff, L)])
    plsc.subcore_barrier()
    poff = pl.multiple_of((s + 1 - 2 * h) * L, 8)
    pltpu.sync_copy(shv.at[pl.ds(poff, L)], pvbuf)
    pltpu.sync_copy(shi.at[pl.ds(poff, L)], pibuf)
    pv = pvbuf[...]
    pi = pibuf[...]

    better = pv > mv
    tie = pv == mv
    toki = jnp.where(better, pi, jnp.where(tie, jnp.minimum(pi, mi), mi))
    tokbuf[...] = toki
    pltpu.sync_copy(tokbuf, out_hbm.at[pl.ds(woff, L)])


def kernel(logits, temperatures):
    wts = jnp.where(temperatures == 0, 0.0, temperatures).astype(jnp.float32)
    gidx = (jnp.arange(NC * NS) // NS) * 8 + (jnp.arange(NC * NS) % NS) // 2
    params = jnp.concatenate(
        [wts.reshape(NG, 8)[gidx], jnp.zeros((NC * NS, 8), jnp.float32)],
        axis=1).reshape(-1)
    out = _sampler(logits, params, _G_OP)
    return out.reshape(NC, 8, 2, L)[:, :, 0, :8].reshape(B)


# speedup vs baseline: 1.2401x; 1.0070x over previous
import functools

import numpy as np

import jax
import jax.numpy as jnp
from jax import lax
from jax.experimental import pallas as pl
from jax.experimental.pallas import tpu as pltpu
from jax.experimental.pallas import tpu_sc as plsc

B = 128
V = 100000
L = 16
NC, NS = 2, 16
NG = B // 8
TILE = 128
HTILES = 390
HCOLS = HTILES * TILE
TAIL0 = 2 * HCOLS
TAILC = V - TAIL0
CT = 10
CW = CT * TILE
NCHUNK = HTILES // CT
CIT = CW // L
TIT = TAILC // L


def _threefry2x32(k1, k2, x0, x1):
    u32 = np.uint32
    R0 = (13, 15, 26, 6)
    R1 = (17, 29, 16, 24)
    ks = (u32(k1), u32(k2), u32(k1) ^ u32(k2) ^ u32(0x1BD11BDA))
    x0 = (x0 + ks[0]).astype(u32)
    x1 = (x1 + ks[1]).astype(u32)

    def rounds(x0, x1, rs):
        for r in rs:
            x0 = (x0 + x1).astype(u32)
            x1 = ((x1 << u32(r)) | (x1 >> u32(32 - r))).astype(u32) ^ x0
        return x0, x1

    for i, (rs, a, b) in enumerate(
            [(R0, 1, 2), (R1, 2, 0), (R0, 0, 1), (R1, 1, 2), (R0, 2, 0)]):
        x0, x1 = rounds(x0, x1, rs)
        x0 = (x0 + ks[a]).astype(u32)
        x1 = (x1 + ks[b] + u32(i + 1)).astype(u32)
    return x0, x1


def _gumbel_const():
    n = B * V
    idx = np.arange(n, dtype=np.uint64)
    hi = (idx >> np.uint64(32)).astype(np.uint32)
    lo = (idx & np.uint64(0xFFFFFFFF)).astype(np.uint32)
    b1, b2 = _threefry2x32(np.uint32(0), np.uint32(42), hi, lo)
    bits = b1 ^ b2
    fb = (bits >> np.uint32(9)) | np.float32(1.0).view(np.uint32)
    u = fb.view(np.float32) - np.float32(1.0)
    noise = (-np.log1p(-u.astype(np.float64))).astype(np.float32)
    noise = np.maximum(noise, np.float32(1e-10))
    g = np.log(noise.astype(np.float64)).astype(np.float32)
    g = g.reshape(B, V)
    ntiles = (V + TILE - 1) // TILE
    gp = np.zeros((B, ntiles * TILE), np.float32)
    gp[:, :V] = g
    return np.ascontiguousarray(
        gp.reshape(NG, 8, ntiles, TILE).transpose(0, 2, 1, 3))


_G = _gumbel_const()

try:
    _G_OP = jax.new_ref(jnp.asarray(_G))
except Exception:
    _G_OP = _G

_mesh = plsc.VectorSubcoreMesh(core_axis_name="c", subcore_axis_name="s")

VPAD = (V + TILE - 1) // TILE * TILE

def _pad_logits(logits):
    return jnp.pad(logits, ((0, 0), (0, VPAD - V)))


@functools.partial(
    pl.kernel,
    out_type=jax.ShapeDtypeStruct((NC * NS * L,), jnp.int32),
    mesh=_mesh,
    compiler_params=pltpu.CompilerParams(needs_layout_passes=False),
    scratch_types=[
        pltpu.VMEM((6, 8, CW), jnp.float32),
        pltpu.VMEM((6, CT, 8, TILE), jnp.float32),
        pltpu.VMEM((8, TAILC), jnp.float32),
        pltpu.VMEM((2, 8, TILE), jnp.float32),
        pltpu.VMEM((L,), jnp.float32),
        pltpu.VMEM((L,), jnp.float32),
        pltpu.VMEM((L,), jnp.int32),
        pltpu.VMEM((L,), jnp.float32),
        pltpu.VMEM((L,), jnp.int32),
        pltpu.VMEM((L,), jnp.int32),
        pltpu.VMEM_SHARED((NS * L,), jnp.float32),
        pltpu.VMEM_SHARED((NS * L,), jnp.int32),
        pltpu.SemaphoreType.DMA,
        pltpu.SemaphoreType.DMA,
        pltpu.SemaphoreType.DMA,
        pltpu.SemaphoreType.DMA,
        pltpu.SemaphoreType.DMA,
        pltpu.SemaphoreType.DMA,
        pltpu.SemaphoreType.DMA,
    ],
)
def _sampler(logits_hbm, params_hbm, g_hbm, out_hbm,
             lbuf, gbuf, ltail, gtail, pbuf, mvbuf, mibuf, pvbuf, pibuf,
             tokbuf, shv, shi, sem0, sem1, sem2, sem3, sem4, sem5, sems):
    c = lax.axis_index("c")
    s = lax.axis_index("s")
    w = c * NS + s
    g = c * 8 + s // 2
    h = s % 2
    row0 = pl.multiple_of(g * 8, 8)
    col_h = pl.multiple_of(h * HCOLS, TILE)
    semslot = (sem0, sem1, sem2, sem3, sem4, sem5)

    woff = pl.multiple_of(w * L, 8)
    pltpu.sync_copy(params_hbm.at[pl.ds(woff, L)], pbuf)
    pvec = pbuf[...]
    wv = [jnp.full((L,), pvec[r], jnp.float32) for r in range(8)]

    lanes = lax.iota(jnp.int32, L)

    tile_h = h * HTILES

    def start(chunk, slot):
        sem = semslot[slot]
        cl = pltpu.async_copy(
            logits_hbm.at[pl.ds(row0, 8), pl.ds(col_h + chunk * CW, CW)],
            lbuf.at[slot], sem)
        cg = pltpu.async_copy(
            g_hbm.at[g, pl.ds(tile_h + chunk * CT, CT)], gbuf.at[slot], sem)
        return cl, cg

    tl = pltpu.async_copy(
        logits_hbm.at[pl.ds(row0, 8), pl.ds(TAIL0, TAILC)], ltail, sems)
    tg = pltpu.async_copy(
        g_hbm.at[g, pl.ds(2 * HTILES, 2)], gtail, sems)

    best = [jnp.full((L,), -jnp.inf, jnp.float32) for _ in range(8)]
    bidx = [jnp.zeros((L,), jnp.int32) for _ in range(8)]

    def make_body(lref, gref, colbase):
        def body(i, carry):
            bs_ = list(carry[:8])
            bi_ = list(carry[8:])
            t = i >> 3
            joff = (i & 7) * L
            off = i * L
            idx = lanes + (colbase + off)
            for r in range(8):
                v = lref[r, pl.ds(off, L)]
                gg = gref[t, r, pl.ds(joff, L)]
                u = v - gg * wv[r]
                m = u > bs_[r]
                bs_[r] = jnp.where(m, u, bs_[r])
                bi_[r] = jnp.where(m, idx, bi_[r])
            return tuple(bs_) + tuple(bi_)
        return body

    NBUF = 6
    pend = [start(k, k) for k in range(NBUF - 1)]
    for chunk in range(NCHUNK):
        slot = chunk % NBUF
        cl, cg = pend.pop(0)
        nxt = chunk + NBUF - 1
        if nxt < NCHUNK:
            pend.append(start(nxt, nxt % NBUF))
        cl.wait()
        cg.wait()
        carry = lax.fori_loop(
            0, CIT, make_body(lbuf.at[slot], gbuf.at[slot], col_h + chunk * CW),
            tuple(best) + tuple(bidx))
        best, bidx = list(carry[:8]), list(carry[8:])

    tl.wait()
    tg.wait()
    carry = tuple(best) + tuple(bidx)
    bs_ = list(carry[:8])
    bi_ = list(carry[8:])
    for i in range(TIT):
        t, j = divmod(i, 8)
        idx = lanes + (TAIL0 + i * L)
        for r in range(8):
            v = ltail[r, pl.ds(i * L, L)]
            gg = gtail[t, r, pl.ds(j * L, L)]
            u = v - gg * wv[r]
            m = u > bs_[r]
            bs_[r] = jnp.where(m, u, bs_[r])
            bi_[r] = jnp.where(m, idx, bi_[r])
    best, bidx = bs_, bi_

    mv = jnp.zeros((L,), jnp.float32)
    mi = jnp.zeros((L,), jnp.int32)
    for r in range(8):
        m = jnp.max(best[r])
        tok = jnp.min(jnp.where(best[r] == m, bidx[r], jnp.int32(2**31 - 1)))
        mv = jnp.where(lanes == r, m, mv)
        mi = jnp.where(lanes == r, tok, mi)
    mvbuf[...] = mv
    mibuf[...] = mi

    soff = pl.multiple_of(s * L, 8)
    pltpu.sync_copy(mvbuf, shv.at[pl.ds(soff, L)])
    pltpu.sync_copy(mibuf, shi.at[pl.ds(soff, L)])
    plsc.subcore_barrier()
    poff = pl.multiple_of((s + 1 - 2 * h) * L, 8)
    pltpu.sync_copy(shv.at[pl.ds(poff, L)], pvbuf)
    pltpu.sync_copy(shi.at[pl.ds(poff, L)], pibuf)
    pv = pvbuf[...]
    pi = pibuf[...]

    better = pv > mv
    tie = pv == mv
    toki = jnp.where(better, pi, jnp.where(tie, jnp.minimum(pi, mi), mi))
    tokbuf[...] = toki
    pltpu.sync_copy(tokbuf, out_hbm.at[pl.ds(woff, L)])


def kernel(logits, temperatures):
    wts = jnp.where(temperatures == 0, 0.0, temperatures).astype(jnp.float32)
    gidx = (jnp.arange(NC * NS) // NS) * 8 + (jnp.arange(NC * NS) % NS) // 2
    params = jnp.concatenate(
        [wts.reshape(NG, 8)[gidx], jnp.zeros((NC * NS, 8), jnp.float32)],
        axis=1).reshape(-1)
    out = _sampler(logits, params, _G_OP)
    return out.reshape(NC, 8, 2, L)[:, :, 0, :8].reshape(B)


# speedup vs baseline: 1.2407x; 1.0004x over previous
import functools

import numpy as np

import jax
import jax.numpy as jnp
from jax import lax
from jax.experimental import pallas as pl
from jax.experimental.pallas import tpu as pltpu
from jax.experimental.pallas import tpu_sc as plsc

B = 128
V = 100000
L = 16
NC, NS = 2, 16
NG = B // 8
TILE = 128
HTILES = 390
HCOLS = HTILES * TILE
TAIL0 = 2 * HCOLS
TAILC = V - TAIL0
CT = 10
CW = CT * TILE
NCHUNK = HTILES // CT
CIT = CW // L
TIT = TAILC // L


def _threefry2x32(k1, k2, x0, x1):
    u32 = np.uint32
    R0 = (13, 15, 26, 6)
    R1 = (17, 29, 16, 24)
    ks = (u32(k1), u32(k2), u32(k1) ^ u32(k2) ^ u32(0x1BD11BDA))
    x0 = (x0 + ks[0]).astype(u32)
    x1 = (x1 + ks[1]).astype(u32)

    def rounds(x0, x1, rs):
        for r in rs:
            x0 = (x0 + x1).astype(u32)
            x1 = ((x1 << u32(r)) | (x1 >> u32(32 - r))).astype(u32) ^ x0
        return x0, x1

    for i, (rs, a, b) in enumerate(
            [(R0, 1, 2), (R1, 2, 0), (R0, 0, 1), (R1, 1, 2), (R0, 2, 0)]):
        x0, x1 = rounds(x0, x1, rs)
        x0 = (x0 + ks[a]).astype(u32)
        x1 = (x1 + ks[b] + u32(i + 1)).astype(u32)
    return x0, x1


def _gumbel_const():
    n = B * V
    idx = np.arange(n, dtype=np.uint64)
    hi = (idx >> np.uint64(32)).astype(np.uint32)
    lo = (idx & np.uint64(0xFFFFFFFF)).astype(np.uint32)
    b1, b2 = _threefry2x32(np.uint32(0), np.uint32(42), hi, lo)
    bits = b1 ^ b2
    fb = (bits >> np.uint32(9)) | np.float32(1.0).view(np.uint32)
    u = fb.view(np.float32) - np.float32(1.0)
    noise = (-np.log1p(-u.astype(np.float64))).astype(np.float32)
    noise = np.maximum(noise, np.float32(1e-10))
    g = np.log(noise.astype(np.float64)).astype(np.float32)
    g = g.reshape(B, V)
    ntiles = (V + TILE - 1) // TILE
    gp = np.zeros((B, ntiles * TILE), np.float32)
    gp[:, :V] = g
    return np.ascontiguousarray(
        gp.reshape(NG, 8, ntiles, TILE).transpose(0, 2, 1, 3))


_G = _gumbel_const()

try:
    _G_OP = jax.new_ref(jnp.asarray(_G))
except Exception:
    _G_OP = _G

_mesh = plsc.VectorSubcoreMesh(core_axis_name="c", subcore_axis_name="s")


@functools.partial(
    pl.kernel,
    out_type=jax.ShapeDtypeStruct((NC * NS * L,), jnp.int32),
    mesh=_mesh,
    compiler_params=pltpu.CompilerParams(needs_layout_passes=False),
    scratch_types=[
        pltpu.VMEM((6, 8, CW), jnp.float32),
        pltpu.VMEM((6, CT, 8, TILE), jnp.float32),
        pltpu.VMEM((8, TAILC), jnp.float32),
        pltpu.VMEM((2, 8, TILE), jnp.float32),
        pltpu.VMEM((L,), jnp.float32),
        pltpu.VMEM((L,), jnp.float32),
        pltpu.VMEM((L,), jnp.int32),
        pltpu.VMEM((L,), jnp.float32),
        pltpu.VMEM((L,), jnp.int32),
        pltpu.VMEM((L,), jnp.int32),
        pltpu.VMEM_SHARED((NS * L,), jnp.float32),
        pltpu.VMEM_SHARED((NS * L,), jnp.int32),
        pltpu.SemaphoreType.DMA,
        pltpu.SemaphoreType.DMA,
        pltpu.SemaphoreType.DMA,
        pltpu.SemaphoreType.DMA,
        pltpu.SemaphoreType.DMA,
        pltpu.SemaphoreType.DMA,
        pltpu.SemaphoreType.DMA,
    ],
)
def _sampler(logits_hbm, params_hbm, g_hbm, out_hbm,
             lbuf, gbuf, ltail, gtail, pbuf, mvbuf, mibuf, pvbuf, pibuf,
             tokbuf, shv, shi, sem0, sem1, sem2, sem3, sem4, sem5, sems):
    c = lax.axis_index("c")
    s = lax.axis_index("s")
    w = c * NS + s
    g = c * 8 + s // 2
    h = s % 2
    row0 = pl.multiple_of(g * 8, 8)
    col_h = pl.multiple_of(h * HCOLS, TILE)
    semslot = (sem0, sem1, sem2, sem3, sem4, sem5)

    woff = pl.multiple_of(w * L, 8)
    pltpu.sync_copy(params_hbm.at[pl.ds(woff, L)], pbuf)
    pvec = pbuf[...]
    wv = [jnp.full((L,), pvec[r], jnp.float32) for r in range(8)]

    lanes = lax.iota(jnp.int32, L)

    tile_h = h * HTILES

    def start(chunk, slot):
        sem = semslot[slot]
        cl = pltpu.async_copy(
            logits_hbm.at[pl.ds(row0, 8), pl.ds(col_h + chunk * CW, CW)],
            lbuf.at[slot], sem)
        cg = pltpu.async_copy(
            g_hbm.at[g, pl.ds(tile_h + chunk * CT, CT)], gbuf.at[slot], sem)
        return cl, cg

    tl = pltpu.async_copy(
        logits_hbm.at[pl.ds(row0, 8), pl.ds(TAIL0, TAILC)], ltail, sems)
    tg = pltpu.async_copy(
        g_hbm.at[g, pl.ds(2 * HTILES, 2)], gtail, sems)

    best = [jnp.full((L,), -jnp.inf, jnp.float32) for _ in range(8)]
    bidx = [jnp.zeros((L,), jnp.int32) for _ in range(8)]

    def make_body(lref, gref, colbase):
        def body(i, carry):
            bs_ = list(carry[:8])
            bi_ = list(carry[8:])
            t = i >> 3
            joff = (i & 7) * L
            off = i * L
            idx = lanes + (colbase + off)
            for r in range(8):
                v = lref[r, pl.ds(off, L)]
                gg = gref[t, r, pl.ds(joff, L)]
                u = v - gg * wv[r]
                m = u > bs_[r]
                bs_[r] = jnp.where(m, u, bs_[r])
                bi_[r] = jnp.where(m, idx, bi_[r])
            return tuple(bs_) + tuple(bi_)
        return body

    NBUF = 6
    pend = [start(k, k) for k in range(NBUF - 1)]
    for chunk in range(NCHUNK):
        slot = chunk % NBUF
        cl, cg = pend.pop(0)
        nxt = chunk + NBUF - 1
        if nxt < NCHUNK:
            pend.append(start(nxt, nxt % NBUF))
        cl.wait()
        cg.wait()
        carry = lax.fori_loop(
            0, CIT, make_body(lbuf.at[slot], gbuf.at[slot], col_h + chunk * CW),
            tuple(best) + tuple(bidx))
        best, bidx = list(carry[:8]), list(carry[8:])

    tl.wait()
    tg.wait()
    carry = tuple(best) + tuple(bidx)
    bs_ = list(carry[:8])
    bi_ = list(carry[8:])
    for i in range(TIT):
        t, j = divmod(i, 8)
        idx = lanes + (TAIL0 + i * L)
        for r in range(8):
            v = ltail[r, pl.ds(i * L, L)]
            gg = gtail[t, r, pl.ds(j * L, L)]
            u = v - gg * wv[r]
            m = u > bs_[r]
            bs_[r] = jnp.where(m, u, bs_[r])
            bi_[r] = jnp.where(m, idx, bi_[r])
    best, bidx = bs_, bi_

    mv = jnp.zeros((L,), jnp.float32)
    mi = jnp.zeros((L,), jnp.int32)
    for r in range(8):
        m = jnp.max(best[r])
        tok = jnp.min(jnp.where(best[r] == m, bidx[r], jnp.int32(2**31 - 1)))
        mv = jnp.where(lanes == r, m, mv)
        mi = jnp.where(lanes == r, tok, mi)
    mvbuf[...] = mv
    mibuf[...] = mi

    soff = pl.multiple_of(s * L, 8)
    pltpu.sync_copy(mvbuf, shv.at[pl.ds(soff, L)])
    pltpu.sync_copy(mibuf, shi.at[pl.ds(soff, L)])
    plsc.subcore_barrier()
    poff = pl.multiple_of((s + 1 - 2 * h) * L, 8)
    pltpu.sync_copy(shv.at[pl.ds(poff, L)], pvbuf)
    pltpu.sync_copy(shi.at[pl.ds(poff, L)], pibuf)
    pv = pvbuf[...]
    pi = pibuf[...]

    better = pv > mv
    tie = pv == mv
    toki = jnp.where(better, pi, jnp.where(tie, jnp.minimum(pi, mi), mi))
    tokbuf[...] = toki
    pltpu.sync_copy(tokbuf, out_hbm.at[pl.ds(woff, L)])


def kernel(logits, temperatures):
    wts = jnp.where(temperatures == 0, 0.0, temperatures).astype(jnp.float32)
    gidx = (jnp.arange(NC * NS) // NS) * 8 + (jnp.arange(NC * NS) % NS) // 2
    params = jnp.concatenate(
        [wts.reshape(NG, 8)[gidx], jnp.zeros((NC * NS, 8), jnp.float32)],
        axis=1).reshape(-1)
    out = _sampler(logits, params, _G_OP)
    return out.reshape(NC, 8, 2, L)[:, :, 0, :8].reshape(B)
